# baseline (speedup 1.0000x reference)
import functools

import jax
import jax.numpy as jnp
from jax import lax
from jax.experimental import pallas as pl
from jax.experimental.pallas import tpu as pltpu
from jax.experimental.pallas import tpu_sc as plsc

B = 16
N = 2048
S = N + 2
D = 256
L = 16
NVEC = D // L

TROWS = 112

HALF = 1024
NBLK = HALF // L
NPAD = 8 + N + 24

_MESH = plsc.VectorSubcoreMesh(
    core_axis_name="c", subcore_axis_name="s", num_cores=2, num_subcores=16
)


@functools.partial(
    pl.kernel,
    out_type=jax.ShapeDtypeStruct((B, S, D), jnp.float32),
    mesh=_MESH,
    scratch_types=[
        pltpu.VMEM((TROWS * D,), jnp.float32),
        pltpu.VMEM((NPAD,), jnp.int32),
        pltpu.VMEM((NPAD,), jnp.int32),
        pltpu.VMEM((NPAD,), jnp.int32),
        pltpu.VMEM((B,), jnp.int32),
        pltpu.VMEM((D,), jnp.float32),
        pltpu.VMEM((D,), jnp.float32),
        pltpu.VMEM((2, L, D), jnp.float32),
        pltpu.SemaphoreType.DMA,
        pltpu.SemaphoreType.DMA,
    ],
    compiler_params=pltpu.CompilerParams(needs_layout_passes=False),
)
def _sc_embed(pT_hbm, eta_hbm, phi_hbm, cnt_hbm, tab_hbm, gam_hbm, bet_hbm,
              out_hbm, tab_v, pT_v, eta_v, phi_v, cnt_v, gam_v, bet_v,
              buf_v, sem0, sem1):
    wid = lax.axis_index("s") * 2 + lax.axis_index("c")
    b = wid // 2
    half = wid % 2
    s_lo = half * HALF

    zeros16 = jnp.zeros((L,), jnp.int32)
    for arr in (pT_v, eta_v, phi_v):
        arr[pl.ds(0, L)] = zeros16
        arr[pl.ds(8 + N, L)] = zeros16
        arr[pl.ds(NPAD - L, L)] = zeros16
    pltpu.sync_copy(tab_hbm, tab_v)
    pltpu.sync_copy(pT_hbm.at[pl.ds(b * N, N)], pT_v.at[pl.ds(8, N)])
    pltpu.sync_copy(eta_hbm.at[pl.ds(b * N, N)], eta_v.at[pl.ds(8, N)])
    pltpu.sync_copy(phi_hbm.at[pl.ds(b * N, N)], phi_v.at[pl.ds(8, N)])
    pltpu.sync_copy(cnt_hbm, cnt_v)
    pltpu.sync_copy(gam_hbm, gam_v)
    pltpu.sync_copy(bet_hbm, bet_v)

    lane = lax.iota(jnp.int32, L)

    def _bcast(v, i):
        return v.at[jnp.full((L,), i, jnp.int32)].get(
            mode="promise_in_bounds")

    cnt_vec = cnt_v[pl.ds(0, L)]
    cb = _bcast(cnt_vec, b)
    sems = (sem0, sem1)

    gs = [gam_v[pl.ds(j * L, L)] for j in range(NVEC)]
    bs = [bet_v[pl.ds(j * L, L)] for j in range(NVEC)]

    def do_row(off0v, off1v, off2v, ri, kb):
        b0 = _bcast(off0v, ri) + lane
        b1 = _bcast(off1v, ri) + lane
        b2 = _bcast(off2v, ri) + lane

        es = []
        for j in range(NVEC):
            t0 = plsc.load_gather(tab_v, [b0 + j * L])
            t1 = plsc.load_gather(tab_v, [b1 + j * L])
            t2 = plsc.load_gather(tab_v, [b2 + j * L])
            es.append(t0 + t1 + t2)

        sv = es
        qv = [e * e for e in es]
        while len(sv) > 1:
            sv = [sv[i] + sv[i + 1] for i in range(0, len(sv), 2)]
        while len(qv) > 1:
            qv = [qv[i] + qv[i + 1] for i in range(0, len(qv), 2)]
        mv = _bcast(plsc.cumsum(sv[0]), 15) * (1.0 / D)
        totq_v = _bcast(plsc.cumsum(qv[0]), 15)
        var_v = totq_v * (1.0 / D) - mv * mv

        xv = var_v + 1e-5
        seed = jnp.full((L,), 0x5F3759DF, dtype=jnp.int32)
        y = plsc.bitcast(seed - (plsc.bitcast(xv, jnp.int32) >> 1), jnp.float32)
        half_x = xv * 0.5
        for _ in range(2):
            y = y * (1.5 - half_x * y * y)

        for j in range(NVEC):
            buf_v[kb, ri, pl.ds(j * L, L)] = (es[j] - mv) * y * gs[j] + bs[j]

    def do_block(t_abs, kb, nrows):
        svec = t_abs + lane
        mask_le = svec <= cb
        is_start = svec == 0
        is_stop = svec == cb + 1
        special = is_start | is_stop
        w2 = [arr[pl.ds(t_abs + 6, L)] for arr in (pT_v, eta_v, phi_v)]
        w1 = [arr[pl.ds(t_abs + 7, L)] for arr in (pT_v, eta_v, phi_v)]
        sel = [jnp.where(mask_le, a, bb) for a, bb in zip(w1, w2)]
        r0 = jnp.clip(sel[0] + 1, 0, 41)
        r0 = jnp.where(special, jnp.where(is_start, 106, 107), r0)
        r1 = jnp.where(special, 108, 42 + jnp.clip(sel[1] + 1, 0, 31))
        r2 = jnp.where(special, 109, 74 + jnp.clip(sel[2] + 1, 0, 31))
        off0 = r0 * D
        off1 = r1 * D
        off2 = r2 * D

        @plsc.parallel_loop(0, nrows, 1, unroll=2)
        def row_body(ri):
            do_row(off0, off1, off2, ri, kb)

    def blk_body(k, _):
        kb = k % 2
        t_abs = s_lo + k * L

        for kbi in (0, 1):
            @pl.when((k >= 2) & (kb == kbi))
            def _wait_prev(kbi=kbi):
                pltpu.make_async_copy(
                    buf_v.at[kbi], out_hbm.at[b, pl.ds(t_abs, L)], sems[kbi]
                ).wait()

        do_block(t_abs, kb, L)
        for kbi in (0, 1):
            @pl.when(kb == kbi)
            def _issue(kbi=kbi):
                pltpu.async_copy(
                    buf_v.at[kbi], out_hbm.at[b, pl.ds(t_abs, L)], sems[kbi]
                )
        return _

    lax.fori_loop(0, NBLK, blk_body, 0)
    pltpu.make_async_copy(
        buf_v.at[0], out_hbm.at[b, pl.ds(s_lo, L)], sems[0]
    ).wait()
    pltpu.make_async_copy(
        buf_v.at[1], out_hbm.at[b, pl.ds(s_lo, L)], sems[1]
    ).wait()

    @pl.when(half == 1)
    def _tail():
        do_block(jnp.int32(N), 0, 2)
        pltpu.sync_copy(buf_v.at[0, pl.ds(0, 2)], out_hbm.at[b, pl.ds(N, 2)])


def kernel(pT_bins, eta_bins, phi_bins, counts, pT_table, eta_table,
           phi_table, start_token, stop_token, gamma, beta):
    pT = pT_bins.astype(jnp.int32).reshape(-1)
    eta = eta_bins.astype(jnp.int32).reshape(-1)
    phi = phi_bins.astype(jnp.int32).reshape(-1)
    cnt = counts.astype(jnp.int32)
    tab = jnp.concatenate(
        [
            pT_table.astype(jnp.float32),
            eta_table.astype(jnp.float32),
            phi_table.astype(jnp.float32),
            start_token[None].astype(jnp.float32),
            stop_token[None].astype(jnp.float32),
            jnp.zeros((TROWS - 108, D), jnp.float32),
        ],
        axis=0,
    ).reshape(-1)
    return _sc_embed(pT, eta, phi, cnt, tab, gamma.astype(jnp.float32),
                     beta.astype(jnp.float32))


# speedup vs baseline: 1.1453x; 1.1453x over previous
import functools

import jax
import jax.numpy as jnp
from jax import lax
from jax.experimental import pallas as pl
from jax.experimental.pallas import tpu as pltpu
from jax.experimental.pallas import tpu_sc as plsc

B = 16
N = 2048
S = N + 2
D = 256
L = 16
NVEC = D // L

TROWS = 112

HALF = 1024
NBLK = HALF // L
NPAD = 8 + N + 24

_MESH = plsc.VectorSubcoreMesh(
    core_axis_name="c", subcore_axis_name="s", num_cores=2, num_subcores=16
)


@functools.partial(
    pl.kernel,
    out_type=jax.ShapeDtypeStruct((B, S, D), jnp.float32),
    mesh=_MESH,
    scratch_types=[
        pltpu.VMEM((TROWS * D,), jnp.float32),
        pltpu.VMEM((NPAD,), jnp.int32),
        pltpu.VMEM((NPAD,), jnp.int32),
        pltpu.VMEM((NPAD,), jnp.int32),
        pltpu.VMEM((B,), jnp.int32),
        pltpu.VMEM((D,), jnp.float32),
        pltpu.VMEM((D,), jnp.float32),
        pltpu.VMEM((2, L, D), jnp.float32),
        pltpu.SemaphoreType.DMA,
        pltpu.SemaphoreType.DMA,
    ],
    compiler_params=pltpu.CompilerParams(needs_layout_passes=False),
)
def _sc_embed(pT_hbm, eta_hbm, phi_hbm, cnt_hbm, tab_hbm, gam_hbm, bet_hbm,
              out_hbm, tab_v, pT_v, eta_v, phi_v, cnt_v, gam_v, bet_v,
              buf_v, sem0, sem1):
    wid = lax.axis_index("s") * 2 + lax.axis_index("c")
    b = wid // 2
    half = wid % 2
    s_lo = half * HALF

    zeros16 = jnp.zeros((L,), jnp.int32)
    for arr in (pT_v, eta_v, phi_v):
        arr[pl.ds(0, L)] = zeros16
        arr[pl.ds(8 + N, L)] = zeros16
        arr[pl.ds(NPAD - L, L)] = zeros16
    pltpu.sync_copy(tab_hbm, tab_v)
    pltpu.sync_copy(pT_hbm.at[pl.ds(b * N, N)], pT_v.at[pl.ds(8, N)])
    pltpu.sync_copy(eta_hbm.at[pl.ds(b * N, N)], eta_v.at[pl.ds(8, N)])
    pltpu.sync_copy(phi_hbm.at[pl.ds(b * N, N)], phi_v.at[pl.ds(8, N)])
    pltpu.sync_copy(cnt_hbm, cnt_v)
    pltpu.sync_copy(gam_hbm, gam_v)
    pltpu.sync_copy(bet_hbm, bet_v)

    lane = lax.iota(jnp.int32, L)

    def _bcast(v, i):
        return v.at[jnp.full((L,), i, jnp.int32)].get(
            mode="promise_in_bounds")

    cnt_vec = cnt_v[pl.ds(0, L)]
    cb = _bcast(cnt_vec, b)
    sems = (sem0, sem1)


    def do_row(off0v, off1v, off2v, ri, kb):
        b0 = _bcast(off0v, ri)[0]
        b1 = _bcast(off1v, ri)[0]
        b2 = _bcast(off2v, ri)[0]

        es = []
        for j in range(NVEC):
            t0 = tab_v[pl.ds(b0 + j * L, L)]
            t1 = tab_v[pl.ds(b1 + j * L, L)]
            t2 = tab_v[pl.ds(b2 + j * L, L)]
            es.append(t0 + t1 + t2)

        sv = es
        qv = [e * e for e in es]
        while len(sv) > 1:
            sv = [sv[i] + sv[i + 1] for i in range(0, len(sv), 2)]
        while len(qv) > 1:
            qv = [qv[i] + qv[i + 1] for i in range(0, len(qv), 2)]
        mv = _bcast(plsc.cumsum(sv[0]), 15) * (1.0 / D)
        totq_v = _bcast(plsc.cumsum(qv[0]), 15)
        var_v = totq_v * (1.0 / D) - mv * mv

        xv = var_v + 1e-5
        seed = jnp.full((L,), 0x5F3759DF, dtype=jnp.int32)
        y = plsc.bitcast(seed - (plsc.bitcast(xv, jnp.int32) >> 1), jnp.float32)
        half_x = xv * 0.5
        for _ in range(2):
            y = y * (1.5 - half_x * y * y)

        for j in range(NVEC):
            g = gam_v[pl.ds(j * L, L)]
            bt = bet_v[pl.ds(j * L, L)]
            buf_v[kb, ri, pl.ds(j * L, L)] = (es[j] - mv) * y * g + bt

    def do_block(t_abs, kb, nrows):
        svec = t_abs + lane
        mask_le = svec <= cb
        is_start = svec == 0
        is_stop = svec == cb + 1
        special = is_start | is_stop
        w2 = [arr[pl.ds(t_abs + 6, L)] for arr in (pT_v, eta_v, phi_v)]
        w1 = [arr[pl.ds(t_abs + 7, L)] for arr in (pT_v, eta_v, phi_v)]
        sel = [jnp.where(mask_le, a, bb) for a, bb in zip(w1, w2)]
        r0 = jnp.clip(sel[0] + 1, 0, 41)
        r0 = jnp.where(special, jnp.where(is_start, 106, 107), r0)
        r1 = jnp.where(special, 108, 42 + jnp.clip(sel[1] + 1, 0, 31))
        r2 = jnp.where(special, 109, 74 + jnp.clip(sel[2] + 1, 0, 31))
        off0 = r0 * D
        off1 = r1 * D
        off2 = r2 * D

        @plsc.parallel_loop(0, nrows, 1, unroll=2)
        def row_body(ri):
            do_row(off0, off1, off2, ri, kb)

    def blk_body(k, _):
        kb = k % 2
        t_abs = s_lo + k * L

        for kbi in (0, 1):
            @pl.when((k >= 2) & (kb == kbi))
            def _wait_prev(kbi=kbi):
                pltpu.make_async_copy(
                    buf_v.at[kbi], out_hbm.at[b, pl.ds(t_abs, L)], sems[kbi]
                ).wait()

        do_block(t_abs, kb, L)
        for kbi in (0, 1):
            @pl.when(kb == kbi)
            def _issue(kbi=kbi):
                pltpu.async_copy(
                    buf_v.at[kbi], out_hbm.at[b, pl.ds(t_abs, L)], sems[kbi]
                )
        return _

    lax.fori_loop(0, NBLK, blk_body, 0)
    pltpu.make_async_copy(
        buf_v.at[0], out_hbm.at[b, pl.ds(s_lo, L)], sems[0]
    ).wait()
    pltpu.make_async_copy(
        buf_v.at[1], out_hbm.at[b, pl.ds(s_lo, L)], sems[1]
    ).wait()

    @pl.when(half == 1)
    def _tail():
        do_block(jnp.int32(N), 0, 2)
        pltpu.sync_copy(buf_v.at[0, pl.ds(0, 2)], out_hbm.at[b, pl.ds(N, 2)])


def kernel(pT_bins, eta_bins, phi_bins, counts, pT_table, eta_table,
           phi_table, start_token, stop_token, gamma, beta):
    pT = pT_bins.astype(jnp.int32).reshape(-1)
    eta = eta_bins.astype(jnp.int32).reshape(-1)
    phi = phi_bins.astype(jnp.int32).reshape(-1)
    cnt = counts.astype(jnp.int32)
    tab = jnp.concatenate(
        [
            pT_table.astype(jnp.float32),
            eta_table.astype(jnp.float32),
            phi_table.astype(jnp.float32),
            start_token[None].astype(jnp.float32),
            stop_token[None].astype(jnp.float32),
            jnp.zeros((TROWS - 108, D), jnp.float32),
        ],
        axis=0,
    ).reshape(-1)
    return _sc_embed(pT, eta, phi, cnt, tab, gamma.astype(jnp.float32),
                     beta.astype(jnp.float32))


# speedup vs baseline: 1.2276x; 1.0719x over previous
import functools

import jax
import jax.numpy as jnp
from jax import lax
from jax.experimental import pallas as pl
from jax.experimental.pallas import tpu as pltpu
from jax.experimental.pallas import tpu_sc as plsc

B = 16
N = 2048
S = N + 2
D = 256
L = 16
NVEC = D // L

TROWS = 112

HALF = 1024
NBLK = HALF // L
NPAD = 8 + N + 24

_MESH = plsc.VectorSubcoreMesh(
    core_axis_name="c", subcore_axis_name="s", num_cores=2, num_subcores=16
)


@functools.partial(
    pl.kernel,
    out_type=jax.ShapeDtypeStruct((B, S, D), jnp.float32),
    mesh=_MESH,
    scratch_types=[
        pltpu.VMEM((TROWS * D,), jnp.float32),
        pltpu.VMEM((NPAD,), jnp.int32),
        pltpu.VMEM((NPAD,), jnp.int32),
        pltpu.VMEM((NPAD,), jnp.int32),
        pltpu.VMEM((B,), jnp.int32),
        pltpu.VMEM((D,), jnp.float32),
        pltpu.VMEM((D,), jnp.float32),
        pltpu.VMEM((2, L, D), jnp.float32),
        pltpu.SemaphoreType.DMA,
        pltpu.SemaphoreType.DMA,
    ],
    compiler_params=pltpu.CompilerParams(needs_layout_passes=False),
)
def _sc_embed(pT_hbm, eta_hbm, phi_hbm, cnt_hbm, tab_hbm, gam_hbm, bet_hbm,
              out_hbm, tab_v, pT_v, eta_v, phi_v, cnt_v, gam_v, bet_v,
              buf_v, sem0, sem1):
    wid = lax.axis_index("s") * 2 + lax.axis_index("c")
    b = wid // 2
    half = wid % 2
    s_lo = half * HALF

    zeros16 = jnp.zeros((L,), jnp.int32)
    for arr in (pT_v, eta_v, phi_v):
        arr[pl.ds(0, L)] = zeros16
        arr[pl.ds(8 + N, L)] = zeros16
        arr[pl.ds(NPAD - L, L)] = zeros16
    pltpu.sync_copy(tab_hbm, tab_v)
    pltpu.sync_copy(pT_hbm.at[pl.ds(b * N, N)], pT_v.at[pl.ds(8, N)])
    pltpu.sync_copy(eta_hbm.at[pl.ds(b * N, N)], eta_v.at[pl.ds(8, N)])
    pltpu.sync_copy(phi_hbm.at[pl.ds(b * N, N)], phi_v.at[pl.ds(8, N)])
    pltpu.sync_copy(cnt_hbm, cnt_v)
    pltpu.sync_copy(gam_hbm, gam_v)
    pltpu.sync_copy(bet_hbm, bet_v)

    lane = lax.iota(jnp.int32, L)

    def _bcast(v, i):
        return v.at[jnp.full((L,), i, jnp.int32)].get(
            mode="promise_in_bounds")

    _perms = [lax.bitwise_xor(lane, jnp.int32(sh)) for sh in (8, 4, 2, 1)]

    def _hsum(v):
        for p in _perms:
            v = v + v.at[p].get(mode="promise_in_bounds")
        return v

    cnt_vec = cnt_v[pl.ds(0, L)]
    cb = _bcast(cnt_vec, b)
    sems = (sem0, sem1)

    gs = [gam_v[pl.ds(j * L, L)] for j in range(NVEC)]
    bs = [bet_v[pl.ds(j * L, L)] for j in range(NVEC)]

    def do_row(off0v, off1v, off2v, ri, kb):
        b0 = _bcast(off0v, ri)[0]
        b1 = _bcast(off1v, ri)[0]
        b2 = _bcast(off2v, ri)[0]

        es = []
        for j in range(NVEC):
            t0 = tab_v[pl.ds(b0 + j * L, L)]
            t1 = tab_v[pl.ds(b1 + j * L, L)]
            t2 = tab_v[pl.ds(b2 + j * L, L)]
            es.append(t0 + t1 + t2)

        sv = es
        qv = [e * e for e in es]
        while len(sv) > 1:
            sv = [sv[i] + sv[i + 1] for i in range(0, len(sv), 2)]
        while len(qv) > 1:
            qv = [qv[i] + qv[i + 1] for i in range(0, len(qv), 2)]
        mv = _hsum(sv[0]) * (1.0 / D)
        var_v = _hsum(qv[0]) * (1.0 / D) - mv * mv

        xv = var_v + 1e-5
        seed = jnp.full((L,), 0x5F3759DF, dtype=jnp.int32)
        y = plsc.bitcast(seed - (plsc.bitcast(xv, jnp.int32) >> 1), jnp.float32)
        half_x = xv * 0.5
        for _ in range(2):
            y = y * (1.5 - half_x * y * y)

        for j in range(NVEC):
            buf_v[kb, ri, pl.ds(j * L, L)] = (es[j] - mv) * y * gs[j] + bs[j]

    def do_block(t_abs, kb, nrows):
        svec = t_abs + lane
        mask_le = svec <= cb
        is_start = svec == 0
        is_stop = svec == cb + 1
        special = is_start | is_stop
        w2 = [arr[pl.ds(t_abs + 6, L)] for arr in (pT_v, eta_v, phi_v)]
        w1 = [arr[pl.ds(t_abs + 7, L)] for arr in (pT_v, eta_v, phi_v)]
        sel = [jnp.where(mask_le, a, bb) for a, bb in zip(w1, w2)]
        r0 = jnp.clip(sel[0] + 1, 0, 41)
        r0 = jnp.where(special, jnp.where(is_start, 106, 107), r0)
        r1 = jnp.where(special, 108, 42 + jnp.clip(sel[1] + 1, 0, 31))
        r2 = jnp.where(special, 109, 74 + jnp.clip(sel[2] + 1, 0, 31))
        off0 = r0 * D
        off1 = r1 * D
        off2 = r2 * D

        @plsc.parallel_loop(0, nrows, 1, unroll=2)
        def row_body(ri):
            do_row(off0, off1, off2, ri, kb)

    def blk_body(k, _):
        kb = k % 2
        t_abs = s_lo + k * L

        for kbi in (0, 1):
            @pl.when((k >= 2) & (kb == kbi))
            def _wait_prev(kbi=kbi):
                pltpu.make_async_copy(
                    buf_v.at[kbi], out_hbm.at[b, pl.ds(t_abs, L)], sems[kbi]
                ).wait()

        do_block(t_abs, kb, L)
        for kbi in (0, 1):
            @pl.when(kb == kbi)
            def _issue(kbi=kbi):
                pltpu.async_copy(
                    buf_v.at[kbi], out_hbm.at[b, pl.ds(t_abs, L)], sems[kbi]
                )
        return _

    lax.fori_loop(0, NBLK, blk_body, 0)
    pltpu.make_async_copy(
        buf_v.at[0], out_hbm.at[b, pl.ds(s_lo, L)], sems[0]
    ).wait()
    pltpu.make_async_copy(
        buf_v.at[1], out_hbm.at[b, pl.ds(s_lo, L)], sems[1]
    ).wait()

    @pl.when(half == 1)
    def _tail():
        do_block(jnp.int32(N), 0, 2)
        pltpu.sync_copy(buf_v.at[0, pl.ds(0, 2)], out_hbm.at[b, pl.ds(N, 2)])


def kernel(pT_bins, eta_bins, phi_bins, counts, pT_table, eta_table,
           phi_table, start_token, stop_token, gamma, beta):
    pT = pT_bins.astype(jnp.int32).reshape(-1)
    eta = eta_bins.astype(jnp.int32).reshape(-1)
    phi = phi_bins.astype(jnp.int32).reshape(-1)
    cnt = counts.astype(jnp.int32)
    tab = jnp.concatenate(
        [
            pT_table.astype(jnp.float32),
            eta_table.astype(jnp.float32),
            phi_table.astype(jnp.float32),
            start_token[None].astype(jnp.float32),
            stop_token[None].astype(jnp.float32),
            jnp.zeros((TROWS - 108, D), jnp.float32),
        ],
        axis=0,
    ).reshape(-1)
    return _sc_embed(pT, eta, phi, cnt, tab, gamma.astype(jnp.float32),
                     beta.astype(jnp.float32))


# speedup vs baseline: 1.2697x; 1.0343x over previous
import functools

import jax
import jax.numpy as jnp
from jax import lax
from jax.experimental import pallas as pl
from jax.experimental.pallas import tpu as pltpu
from jax.experimental.pallas import tpu_sc as plsc

B = 16
N = 2048
S = N + 2
D = 256
L = 16
NVEC = D // L

TROWS = 112

HALF = 1024
NBLK = HALF // L
NPAD = 8 + N + 24

_MESH = plsc.VectorSubcoreMesh(
    core_axis_name="c", subcore_axis_name="s", num_cores=2, num_subcores=16
)


@functools.partial(
    pl.kernel,
    out_type=jax.ShapeDtypeStruct((B, S, D), jnp.float32),
    mesh=_MESH,
    scratch_types=[
        pltpu.VMEM((TROWS * D,), jnp.float32),
        pltpu.VMEM((NPAD,), jnp.int32),
        pltpu.VMEM((NPAD,), jnp.int32),
        pltpu.VMEM((NPAD,), jnp.int32),
        pltpu.VMEM((B,), jnp.int32),
        pltpu.VMEM((D,), jnp.float32),
        pltpu.VMEM((D,), jnp.float32),
        pltpu.VMEM((2, L, D), jnp.float32),
        pltpu.SemaphoreType.DMA,
        pltpu.SemaphoreType.DMA,
    ],
    compiler_params=pltpu.CompilerParams(needs_layout_passes=False),
)
def _sc_embed(pT_hbm, eta_hbm, phi_hbm, cnt_hbm, tab_hbm, gam_hbm, bet_hbm,
              out_hbm, tab_v, pT_v, eta_v, phi_v, cnt_v, gam_v, bet_v,
              buf_v, sem0, sem1):
    wid = lax.axis_index("s") * 2 + lax.axis_index("c")
    b = wid // 2
    half = wid % 2
    s_lo = half * HALF

    zeros16 = jnp.zeros((L,), jnp.int32)
    for arr in (pT_v, eta_v, phi_v):
        arr[pl.ds(0, L)] = zeros16
        arr[pl.ds(8 + N, L)] = zeros16
        arr[pl.ds(NPAD - L, L)] = zeros16
    pltpu.sync_copy(tab_hbm, tab_v)
    pltpu.sync_copy(pT_hbm.at[pl.ds(b * N, N)], pT_v.at[pl.ds(8, N)])
    pltpu.sync_copy(eta_hbm.at[pl.ds(b * N, N)], eta_v.at[pl.ds(8, N)])
    pltpu.sync_copy(phi_hbm.at[pl.ds(b * N, N)], phi_v.at[pl.ds(8, N)])
    pltpu.sync_copy(cnt_hbm, cnt_v)
    pltpu.sync_copy(gam_hbm, gam_v)
    pltpu.sync_copy(bet_hbm, bet_v)

    lane = lax.iota(jnp.int32, L)

    def _bcast(v, i):
        return v.at[jnp.full((L,), i, jnp.int32)].get(
            mode="promise_in_bounds")

    cnt_vec = cnt_v[pl.ds(0, L)]
    cb = _bcast(cnt_vec, b)
    sems = (sem0, sem1)

    gs = [gam_v[pl.ds(j * L, L)] for j in range(NVEC)]
    bs = [bet_v[pl.ds(j * L, L)] for j in range(NVEC)]

    def do_row(off0v, off1v, off2v, ri, kb):
        b0 = _bcast(off0v, ri)[0]
        b1 = _bcast(off1v, ri)[0]
        b2 = _bcast(off2v, ri)[0]

        es = []
        for j in range(NVEC):
            t0 = tab_v[pl.ds(b0 + j * L, L)]
            t1 = tab_v[pl.ds(b1 + j * L, L)]
            t2 = tab_v[pl.ds(b2 + j * L, L)]
            es.append(t0 + t1 + t2)

        sv = es
        qv = [e * e for e in es]
        while len(sv) > 1:
            sv = [sv[i] + sv[i + 1] for i in range(0, len(sv), 2)]
        while len(qv) > 1:
            qv = [qv[i] + qv[i + 1] for i in range(0, len(qv), 2)]
        mv = _bcast(plsc.cumsum(sv[0]), 15) * (1.0 / D)
        totq_v = _bcast(plsc.cumsum(qv[0]), 15)
        var_v = totq_v * (1.0 / D) - mv * mv

        xv = var_v + 1e-5
        seed = jnp.full((L,), 0x5F3759DF, dtype=jnp.int32)
        y = plsc.bitcast(seed - (plsc.bitcast(xv, jnp.int32) >> 1), jnp.float32)
        half_x = xv * 0.5
        for _ in range(2):
            y = y * (1.5 - half_x * y * y)

        for j in range(NVEC):
            buf_v[kb, ri, pl.ds(j * L, L)] = (es[j] - mv) * y * gs[j] + bs[j]

    def do_block(t_abs, kb, nrows):
        svec = t_abs + lane
        mask_le = svec <= cb
        is_start = svec == 0
        is_stop = svec == cb + 1
        special = is_start | is_stop
        w2 = [arr[pl.ds(t_abs + 6, L)] for arr in (pT_v, eta_v, phi_v)]
        w1 = [arr[pl.ds(t_abs + 7, L)] for arr in (pT_v, eta_v, phi_v)]
        sel = [jnp.where(mask_le, a, bb) for a, bb in zip(w1, w2)]
        r0 = jnp.clip(sel[0] + 1, 0, 41)
        r0 = jnp.where(special, jnp.where(is_start, 106, 107), r0)
        r1 = jnp.where(special, 108, 42 + jnp.clip(sel[1] + 1, 0, 31))
        r2 = jnp.where(special, 109, 74 + jnp.clip(sel[2] + 1, 0, 31))
        off0 = r0 * D
        off1 = r1 * D
        off2 = r2 * D

        @plsc.parallel_loop(0, nrows, 1, unroll=2)
        def row_body(ri):
            do_row(off0, off1, off2, ri, kb)

    def blk_body(k, _):
        kb = k % 2
        t_abs = s_lo + k * L

        for kbi in (0, 1):
            @pl.when((k >= 2) & (kb == kbi))
            def _wait_prev(kbi=kbi):
                pltpu.make_async_copy(
                    buf_v.at[kbi], out_hbm.at[b, pl.ds(t_abs, L)], sems[kbi]
                ).wait()

        do_block(t_abs, kb, L)
        for kbi in (0, 1):
            @pl.when(kb == kbi)
            def _issue(kbi=kbi):
                pltpu.async_copy(
                    buf_v.at[kbi], out_hbm.at[b, pl.ds(t_abs, L)], sems[kbi]
                )
        return _

    lax.fori_loop(0, NBLK, blk_body, 0)
    pltpu.make_async_copy(
        buf_v.at[0], out_hbm.at[b, pl.ds(s_lo, L)], sems[0]
    ).wait()
    pltpu.make_async_copy(
        buf_v.at[1], out_hbm.at[b, pl.ds(s_lo, L)], sems[1]
    ).wait()

    @pl.when(half == 1)
    def _tail():
        do_block(jnp.int32(N), 0, 2)
        pltpu.sync_copy(buf_v.at[0, pl.ds(0, 2)], out_hbm.at[b, pl.ds(N, 2)])


def kernel(pT_bins, eta_bins, phi_bins, counts, pT_table, eta_table,
           phi_table, start_token, stop_token, gamma, beta):
    pT = pT_bins.astype(jnp.int32).reshape(-1)
    eta = eta_bins.astype(jnp.int32).reshape(-1)
    phi = phi_bins.astype(jnp.int32).reshape(-1)
    cnt = counts.astype(jnp.int32)
    tab = jnp.concatenate(
        [
            pT_table.astype(jnp.float32),
            eta_table.astype(jnp.float32),
            phi_table.astype(jnp.float32),
            start_token[None].astype(jnp.float32),
            stop_token[None].astype(jnp.float32),
            jnp.zeros((TROWS - 108, D), jnp.float32),
        ],
        axis=0,
    ).reshape(-1)
    return _sc_embed(pT, eta, phi, cnt, tab, gamma.astype(jnp.float32),
                     beta.astype(jnp.float32))


# speedup vs baseline: 1.2806x; 1.0086x over previous
import functools

import jax
import jax.numpy as jnp
from jax import lax
from jax.experimental import pallas as pl
from jax.experimental.pallas import tpu as pltpu
from jax.experimental.pallas import tpu_sc as plsc

B = 16
N = 2048
S = N + 2
D = 256
L = 16
NVEC = D // L

TROWS = 112

HALF = 1024
NBLK = HALF // L
NPAD = 8 + N + 24

_MESH = plsc.VectorSubcoreMesh(
    core_axis_name="c", subcore_axis_name="s", num_cores=2, num_subcores=16
)


@functools.partial(
    pl.kernel,
    out_type=jax.ShapeDtypeStruct((B, S, D), jnp.float32),
    mesh=_MESH,
    scratch_types=[
        pltpu.VMEM((TROWS * D,), jnp.float32),
        pltpu.VMEM((NPAD,), jnp.int32),
        pltpu.VMEM((NPAD,), jnp.int32),
        pltpu.VMEM((NPAD,), jnp.int32),
        pltpu.VMEM((B,), jnp.int32),
        pltpu.VMEM((D,), jnp.float32),
        pltpu.VMEM((D,), jnp.float32),
        pltpu.VMEM((2, L, D), jnp.float32),
        pltpu.SemaphoreType.DMA,
        pltpu.SemaphoreType.DMA,
    ],
    compiler_params=pltpu.CompilerParams(needs_layout_passes=False),
)
def _sc_embed(pT_hbm, eta_hbm, phi_hbm, cnt_hbm, tab_hbm, gam_hbm, bet_hbm,
              out_hbm, tab_v, pT_v, eta_v, phi_v, cnt_v, gam_v, bet_v,
              buf_v, sem0, sem1):
    wid = lax.axis_index("s") * 2 + lax.axis_index("c")
    b = wid // 2
    half = wid % 2
    s_lo = half * HALF

    zeros16 = jnp.zeros((L,), jnp.int32)
    for arr in (pT_v, eta_v, phi_v):
        arr[pl.ds(0, L)] = zeros16
        arr[pl.ds(8 + N, L)] = zeros16
        arr[pl.ds(NPAD - L, L)] = zeros16
    pltpu.sync_copy(tab_hbm, tab_v)
    pltpu.sync_copy(pT_hbm.at[pl.ds(b * N, N)], pT_v.at[pl.ds(8, N)])
    pltpu.sync_copy(eta_hbm.at[pl.ds(b * N, N)], eta_v.at[pl.ds(8, N)])
    pltpu.sync_copy(phi_hbm.at[pl.ds(b * N, N)], phi_v.at[pl.ds(8, N)])
    pltpu.sync_copy(cnt_hbm, cnt_v)
    pltpu.sync_copy(gam_hbm, gam_v)
    pltpu.sync_copy(bet_hbm, bet_v)

    lane = lax.iota(jnp.int32, L)

    def _bcast(v, i):
        return v.at[jnp.full((L,), i, jnp.int32)].get(
            mode="promise_in_bounds")

    cnt_vec = cnt_v[pl.ds(0, L)]
    cb = _bcast(cnt_vec, b)
    sems = (sem0, sem1)

    gs = [gam_v[pl.ds(j * L, L)] for j in range(NVEC)]
    bs = [bet_v[pl.ds(j * L, L)] for j in range(NVEC)]

    def do_row(pkv, ri, kb):
        s = _bcast(pkv, ri)[0]
        b0 = (s & 127) << 8
        b1 = ((s >> 7) & 127) << 8
        b2 = (s >> 14) << 8

        es = []
        for j in range(NVEC):
            t0 = tab_v[pl.ds(b0 + j * L, L)]
            t1 = tab_v[pl.ds(b1 + j * L, L)]
            t2 = tab_v[pl.ds(b2 + j * L, L)]
            es.append(t0 + t1 + t2)

        sv = es
        qv = [e * e for e in es]
        while len(sv) > 1:
            sv = [sv[i] + sv[i + 1] for i in range(0, len(sv), 2)]
        while len(qv) > 1:
            qv = [qv[i] + qv[i + 1] for i in range(0, len(qv), 2)]
        mv = _bcast(plsc.cumsum(sv[0]), 15) * (1.0 / D)
        totq_v = _bcast(plsc.cumsum(qv[0]), 15)
        var_v = totq_v * (1.0 / D) - mv * mv

        xv = var_v + 1e-5
        seed = jnp.full((L,), 0x5F3759DF, dtype=jnp.int32)
        y = plsc.bitcast(seed - (plsc.bitcast(xv, jnp.int32) >> 1), jnp.float32)
        half_x = xv * 0.5
        for _ in range(2):
            y = y * (1.5 - half_x * y * y)

        for j in range(NVEC):
            buf_v[kb, ri, pl.ds(j * L, L)] = (es[j] - mv) * y * gs[j] + bs[j]

    def do_block(t_abs, kb, nrows):
        svec = t_abs + lane
        mask_le = svec <= cb
        is_start = svec == 0
        is_stop = svec == cb + 1
        special = is_start | is_stop
        w2 = [arr[pl.ds(t_abs + 6, L)] for arr in (pT_v, eta_v, phi_v)]
        w1 = [arr[pl.ds(t_abs + 7, L)] for arr in (pT_v, eta_v, phi_v)]
        sel = [jnp.where(mask_le, a, bb) for a, bb in zip(w1, w2)]
        r0 = jnp.clip(sel[0] + 1, 0, 41)
        r0 = jnp.where(special, jnp.where(is_start, 106, 107), r0)
        r1 = jnp.where(special, 108, 42 + jnp.clip(sel[1] + 1, 0, 31))
        r2 = jnp.where(special, 109, 74 + jnp.clip(sel[2] + 1, 0, 31))
        pk = r0 | (r1 << 7) | (r2 << 14)

        @plsc.parallel_loop(0, nrows, 1, unroll=2)
        def row_body(ri):
            do_row(pk, ri, kb)

    def blk_body(k, _):
        kb = k % 2
        t_abs = s_lo + k * L

        for kbi in (0, 1):
            @pl.when((k >= 2) & (kb == kbi))
            def _wait_prev(kbi=kbi):
                pltpu.make_async_copy(
                    buf_v.at[kbi], out_hbm.at[b, pl.ds(t_abs, L)], sems[kbi]
                ).wait()

        do_block(t_abs, kb, L)
        for kbi in (0, 1):
            @pl.when(kb == kbi)
            def _issue(kbi=kbi):
                pltpu.async_copy(
                    buf_v.at[kbi], out_hbm.at[b, pl.ds(t_abs, L)], sems[kbi]
                )
        return _

    lax.fori_loop(0, NBLK, blk_body, 0)
    pltpu.make_async_copy(
        buf_v.at[0], out_hbm.at[b, pl.ds(s_lo, L)], sems[0]
    ).wait()
    pltpu.make_async_copy(
        buf_v.at[1], out_hbm.at[b, pl.ds(s_lo, L)], sems[1]
    ).wait()

    @pl.when(half == 1)
    def _tail():
        do_block(jnp.int32(N), 0, 2)
        pltpu.sync_copy(buf_v.at[0, pl.ds(0, 2)], out_hbm.at[b, pl.ds(N, 2)])


def kernel(pT_bins, eta_bins, phi_bins, counts, pT_table, eta_table,
           phi_table, start_token, stop_token, gamma, beta):
    pT = pT_bins.astype(jnp.int32).reshape(-1)
    eta = eta_bins.astype(jnp.int32).reshape(-1)
    phi = phi_bins.astype(jnp.int32).reshape(-1)
    cnt = counts.astype(jnp.int32)
    tab = jnp.concatenate(
        [
            pT_table.astype(jnp.float32),
            eta_table.astype(jnp.float32),
            phi_table.astype(jnp.float32),
            start_token[None].astype(jnp.float32),
            stop_token[None].astype(jnp.float32),
            jnp.zeros((TROWS - 108, D), jnp.float32),
        ],
        axis=0,
    ).reshape(-1)
    return _sc_embed(pT, eta, phi, cnt, tab, gamma.astype(jnp.float32),
                     beta.astype(jnp.float32))
